# baseline (device time: 7613 ns/iter reference)
import jax
import jax.numpy as jnp
from jax import lax
from jax.experimental import pallas as pl
from jax.experimental.pallas import tpu as pltpu


def kernel(x, pi):
    def body(
        pi_hbm,
        x_hbm,
        out_ref,
        x_vmem,
        comm_ref,
        pi_smem,
        copy_sems,
        send_sem,
        recv_sem,
    ):
        my_x = lax.axis_index("x")
        my_y = lax.axis_index("y")
        my_z = lax.axis_index("z")
        nbr_y = 1 - my_y

        pi_cp = pltpu.make_async_copy(pi_hbm, pi_smem, copy_sems.at[0])
        pi_cp.start()
        x_cp = pltpu.make_async_copy(x_hbm, x_vmem, copy_sems.at[1])
        x_cp.start()

        barrier_sem = pltpu.get_barrier_semaphore()
        pl.semaphore_signal(
            barrier_sem,
            inc=1,
            device_id=(my_x, nbr_y, my_z),
            device_id_type=pl.DeviceIdType.MESH,
        )
        pl.semaphore_wait(barrier_sem, 1)

        pi_cp.wait()
        x_cp.wait()

        dst_y = jnp.where(my_y == 0, pi_smem[0], pi_smem[1])
        is_swap = dst_y != my_y

        comm_ref[...] = x_vmem[...].astype(jnp.bfloat16)

        @pl.when(is_swap)
        def _():
            rdma = pltpu.make_async_remote_copy(
                src_ref=comm_ref,
                dst_ref=out_ref,
                send_sem=send_sem,
                recv_sem=recv_sem,
                device_id=(my_x, dst_y, my_z),
                device_id_type=pl.DeviceIdType.MESH,
            )
            rdma.start()
            rdma.wait()

        @pl.when(jnp.logical_not(is_swap))
        def _():
            cp = pltpu.make_async_copy(comm_ref, out_ref, copy_sems.at[0])
            cp.start()
            cp.wait()

    return pl.pallas_call(
        body,
        out_shape=jax.ShapeDtypeStruct(x.shape, jnp.bfloat16),
        in_specs=[
            pl.BlockSpec(memory_space=pl.ANY),
            pl.BlockSpec(memory_space=pl.ANY),
        ],
        out_specs=pl.BlockSpec(memory_space=pl.ANY),
        scratch_shapes=[
            pltpu.VMEM(x.shape, x.dtype),
            pltpu.VMEM(x.shape, jnp.bfloat16),
            pltpu.SMEM((2,), jnp.int32),
            pltpu.SemaphoreType.DMA((2,)),
            pltpu.SemaphoreType.DMA,
            pltpu.SemaphoreType.DMA,
        ],
        compiler_params=pltpu.CompilerParams(collective_id=0),
    )(pi, x)


# device time: 7608 ns/iter; 1.0007x vs baseline; 1.0007x over previous
import jax
import jax.numpy as jnp
from jax import lax
from jax.experimental import pallas as pl
from jax.experimental.pallas import tpu as pltpu


def kernel(x, pi):
    def body(
        pi_hbm,
        x_hbm,
        out_ref,
        x_vmem,
        comm_ref,
        pi_smem,
        copy_sems,
        send_sem,
        recv_sem,
    ):
        my_x = lax.axis_index("x")
        my_y = lax.axis_index("y")
        my_z = lax.axis_index("z")
        nbr_y = 1 - my_y

        pi_cp = pltpu.make_async_copy(pi_hbm, pi_smem, copy_sems.at[0])
        pi_cp.start()
        x_cp = pltpu.make_async_copy(x_hbm, x_vmem, copy_sems.at[1])
        x_cp.start()

        barrier_sem = pltpu.get_barrier_semaphore()
        pl.semaphore_signal(
            barrier_sem,
            inc=1,
            device_id=(my_x, nbr_y, my_z),
            device_id_type=pl.DeviceIdType.MESH,
        )
        pl.semaphore_wait(barrier_sem, 1)

        pi_cp.wait()
        x_cp.wait()

        dst_y = jnp.where(my_y == 0, pi_smem[0], pi_smem[1])
        is_swap = dst_y != my_y

        comm_ref[...] = x_vmem[...].astype(jnp.bfloat16)

        @pl.when(is_swap)
        def _():
            rdma = pltpu.make_async_remote_copy(
                src_ref=comm_ref,
                dst_ref=out_ref,
                send_sem=send_sem,
                recv_sem=recv_sem,
                device_id=(my_x, dst_y, my_z),
                device_id_type=pl.DeviceIdType.MESH,
            )
            rdma.start()
            rdma.wait()

        @pl.when(jnp.logical_not(is_swap))
        def _():
            cp = pltpu.make_async_copy(comm_ref, out_ref, copy_sems.at[0])
            cp.start()
            cp.wait()

    return pl.pallas_call(
        body,
        out_shape=jax.ShapeDtypeStruct(x.shape, jnp.bfloat16),
        in_specs=[
            pl.BlockSpec(memory_space=pltpu.MemorySpace.HBM),
            pl.BlockSpec(memory_space=pltpu.MemorySpace.HBM),
        ],
        out_specs=pl.BlockSpec(memory_space=pltpu.MemorySpace.HBM),
        scratch_shapes=[
            pltpu.VMEM(x.shape, x.dtype),
            pltpu.VMEM(x.shape, jnp.bfloat16),
            pltpu.SMEM((2,), jnp.int32),
            pltpu.SemaphoreType.DMA((2,)),
            pltpu.SemaphoreType.DMA,
            pltpu.SemaphoreType.DMA,
        ],
        compiler_params=pltpu.CompilerParams(collective_id=0),
    )(pi, x)


# device time: 7200 ns/iter; 1.0574x vs baseline; 1.0567x over previous
import jax
import jax.numpy as jnp
from jax import lax
from jax.experimental import pallas as pl
from jax.experimental.pallas import tpu as pltpu

N_CHUNKS = 2


def kernel(x, pi):
    rows = x.shape[1]
    assert rows % N_CHUNKS == 0
    chunk = rows // N_CHUNKS

    def body(pi_ref, x_ref, out_ref, comm_ref, send_sems, recv_sems):
        my_x = lax.axis_index("x")
        my_y = lax.axis_index("y")
        my_z = lax.axis_index("z")

        dst_y = jnp.where(my_y == 0, pi_ref[0], pi_ref[1])
        is_swap = dst_y != my_y

        @pl.when(is_swap)
        def _():
            rdmas = []
            for h in range(N_CHUNKS):
                sl = pl.ds(h * chunk, chunk)
                comm_ref[0, sl, :] = x_ref[0, sl, :].astype(jnp.bfloat16)
                rdma = pltpu.make_async_remote_copy(
                    src_ref=comm_ref.at[0, sl, :],
                    dst_ref=out_ref.at[0, sl, :],
                    send_sem=send_sems.at[h],
                    recv_sem=recv_sems.at[h],
                    device_id=(my_x, dst_y, my_z),
                    device_id_type=pl.DeviceIdType.MESH,
                )
                rdma.start()
                rdmas.append(rdma)
            for rdma in rdmas:
                rdma.wait()

        @pl.when(jnp.logical_not(is_swap))
        def _():
            out_ref[...] = x_ref[...].astype(jnp.bfloat16)

    return pl.pallas_call(
        body,
        out_shape=jax.ShapeDtypeStruct(x.shape, jnp.bfloat16),
        in_specs=[
            pl.BlockSpec(memory_space=pltpu.SMEM),
            pl.BlockSpec(memory_space=pltpu.VMEM),
        ],
        out_specs=pl.BlockSpec(memory_space=pltpu.VMEM),
        scratch_shapes=[
            pltpu.VMEM(x.shape, jnp.bfloat16),
            pltpu.SemaphoreType.DMA((N_CHUNKS,)),
            pltpu.SemaphoreType.DMA((N_CHUNKS,)),
        ],
        compiler_params=pltpu.CompilerParams(
            skip_device_barrier=True,
        ),
    )(pi, x)


# device time: 6485 ns/iter; 1.1739x vs baseline; 1.1103x over previous
import jax
import jax.numpy as jnp
from jax import lax
from jax.experimental import pallas as pl
from jax.experimental.pallas import tpu as pltpu

N_CHUNKS = 2


def kernel(x, pi):
    rows = x.shape[1]
    assert rows % N_CHUNKS == 0
    chunk = rows // N_CHUNKS

    x = pltpu.with_memory_space_constraint(x, pltpu.MemorySpace.HBM)
    pi = pltpu.with_memory_space_constraint(pi, pltpu.MemorySpace.HBM)

    def body(
        pi_hbm,
        x_hbm,
        out_ref,
        x_vmem,
        comm_ref,
        pi_smem,
        copy_sems,
        send_sems,
        recv_sems,
    ):
        my_x = lax.axis_index("x")
        my_y = lax.axis_index("y")
        my_z = lax.axis_index("z")

        pi_cp = pltpu.make_async_copy(pi_hbm, pi_smem, copy_sems.at[0])
        pi_cp.start()
        x_cp = pltpu.make_async_copy(x_hbm, x_vmem, copy_sems.at[1])
        x_cp.start()

        pi_cp.wait()
        dst_y = jnp.where(my_y == 0, pi_smem[0], pi_smem[1])
        is_swap = dst_y != my_y
        x_cp.wait()

        @pl.when(is_swap)
        def _():
            rdmas = []
            for h in range(N_CHUNKS):
                sl = pl.ds(h * chunk, chunk)
                comm_ref[0, sl, :] = x_vmem[0, sl, :].astype(jnp.bfloat16)
                rdma = pltpu.make_async_remote_copy(
                    src_ref=comm_ref.at[0, sl, :],
                    dst_ref=out_ref.at[0, sl, :],
                    send_sem=send_sems.at[h],
                    recv_sem=recv_sems.at[h],
                    device_id=(my_x, dst_y, my_z),
                    device_id_type=pl.DeviceIdType.MESH,
                )
                rdma.start()
                rdmas.append(rdma)
            for rdma in rdmas:
                rdma.wait()

        @pl.when(jnp.logical_not(is_swap))
        def _():
            out_ref[...] = x_vmem[...].astype(jnp.bfloat16)

    return pl.pallas_call(
        body,
        out_shape=jax.ShapeDtypeStruct(x.shape, jnp.bfloat16),
        in_specs=[
            pl.BlockSpec(memory_space=pltpu.MemorySpace.HBM),
            pl.BlockSpec(memory_space=pltpu.MemorySpace.HBM),
        ],
        out_specs=pl.BlockSpec(memory_space=pltpu.VMEM),
        scratch_shapes=[
            pltpu.VMEM(x.shape, x.dtype),
            pltpu.VMEM(x.shape, jnp.bfloat16),
            pltpu.SMEM((2,), jnp.int32),
            pltpu.SemaphoreType.DMA((2,)),
            pltpu.SemaphoreType.DMA((N_CHUNKS,)),
            pltpu.SemaphoreType.DMA((N_CHUNKS,)),
        ],
        compiler_params=pltpu.CompilerParams(
            skip_device_barrier=True,
        ),
    )(pi, x)


# device time: 6397 ns/iter; 1.1901x vs baseline; 1.0138x over previous
import jax
import jax.numpy as jnp
from jax import lax
from jax.experimental import pallas as pl
from jax.experimental.pallas import tpu as pltpu

N_CHUNKS = 2


def kernel(x, pi):
    rows = x.shape[1]
    assert rows % N_CHUNKS == 0
    chunk = rows // N_CHUNKS

    x = pltpu.with_memory_space_constraint(x, pltpu.MemorySpace.HBM)
    pi = pltpu.with_memory_space_constraint(pi, pltpu.MemorySpace.HBM)

    def body(
        pi_hbm,
        x_hbm,
        out_ref,
        x_vmem,
        comm_ref,
        pi_smem,
        copy_sems,
        send_sems,
        recv_sems,
    ):
        my_x = lax.axis_index("x")
        my_y = lax.axis_index("y")
        my_z = lax.axis_index("z")

        pi_cp = pltpu.make_async_copy(pi_hbm, pi_smem, copy_sems.at[0])
        pi_cp.start()
        x_cps = []
        for h in range(N_CHUNKS):
            sl = pl.ds(h * chunk, chunk)
            x_cp = pltpu.make_async_copy(
                x_hbm.at[0, sl, :], x_vmem.at[0, sl, :], copy_sems.at[1 + h]
            )
            x_cp.start()
            x_cps.append(x_cp)

        pi_cp.wait()
        dst_y = jnp.where(my_y == 0, pi_smem[0], pi_smem[1])
        is_swap = dst_y != my_y

        @pl.when(is_swap)
        def _():
            rdmas = []
            for h in range(N_CHUNKS):
                sl = pl.ds(h * chunk, chunk)
                x_cps[h].wait()
                comm_ref[0, sl, :] = x_vmem[0, sl, :].astype(jnp.bfloat16)
                rdma = pltpu.make_async_remote_copy(
                    src_ref=comm_ref.at[0, sl, :],
                    dst_ref=out_ref.at[0, sl, :],
                    send_sem=send_sems.at[h],
                    recv_sem=recv_sems.at[h],
                    device_id=(my_x, dst_y, my_z),
                    device_id_type=pl.DeviceIdType.MESH,
                )
                rdma.start()
                rdmas.append(rdma)
            for rdma in rdmas:
                rdma.wait()

        @pl.when(jnp.logical_not(is_swap))
        def _():
            for h in range(N_CHUNKS):
                x_cps[h].wait()
            out_ref[...] = x_vmem[...].astype(jnp.bfloat16)

    return pl.pallas_call(
        body,
        out_shape=jax.ShapeDtypeStruct(x.shape, jnp.bfloat16),
        in_specs=[
            pl.BlockSpec(memory_space=pltpu.MemorySpace.HBM),
            pl.BlockSpec(memory_space=pltpu.MemorySpace.HBM),
        ],
        out_specs=pl.BlockSpec(memory_space=pltpu.VMEM),
        scratch_shapes=[
            pltpu.VMEM(x.shape, x.dtype),
            pltpu.VMEM(x.shape, jnp.bfloat16),
            pltpu.SMEM((2,), jnp.int32),
            pltpu.SemaphoreType.DMA((1 + N_CHUNKS,)),
            pltpu.SemaphoreType.DMA((N_CHUNKS,)),
            pltpu.SemaphoreType.DMA((N_CHUNKS,)),
        ],
        compiler_params=pltpu.CompilerParams(
            skip_device_barrier=True,
        ),
    )(pi, x)
